# initial kernel scaffold (unmeasured)
import jax
import jax.numpy as jnp
from jax import lax
from jax.experimental import pallas as pl
from jax.experimental.pallas import tpu as pltpu

B, S_LOC, H, D = 1, 1024, 16, 128
SCALE = D ** -0.5


def kernel(Q, K, V):
    def body(q_ref, k_ref, v_ref, out_ref, kv_rem, send_sems, recv_sems):
        my_x = lax.axis_index("x")
        my_y = lax.axis_index("y")
        my_z = lax.axis_index("z")
        nbr = (my_x, 1 - my_y, my_z)

        barrier = pltpu.get_barrier_semaphore()
        pl.semaphore_signal(
            barrier, inc=1, device_id=nbr, device_id_type=pl.DeviceIdType.MESH
        )
        pl.semaphore_wait(barrier, 1)

        rdma_k = pltpu.make_async_remote_copy(
            src_ref=k_ref,
            dst_ref=kv_rem.at[0],
            send_sem=send_sems.at[0],
            recv_sem=recv_sems.at[0],
            device_id=nbr,
            device_id_type=pl.DeviceIdType.MESH,
        )
        rdma_v = pltpu.make_async_remote_copy(
            src_ref=v_ref,
            dst_ref=kv_rem.at[1],
            send_sem=send_sems.at[1],
            recv_sem=recv_sems.at[1],
            device_id=nbr,
            device_id_type=pl.DeviceIdType.MESH,
        )
        rdma_k.start()
        rdma_v.start()
        rdma_k.wait()
        rdma_v.wait()

        for h in range(H):
            q = q_ref[0, :, h, :]
            k = jnp.concatenate(
                [k_ref[0, :, h, :], kv_rem[0, 0, :, h, :]], axis=0
            )
            v = jnp.concatenate(
                [v_ref[0, :, h, :], kv_rem[1, 0, :, h, :]], axis=0
            )
            s = (
                lax.dot_general(
                    q, k, (((1,), (1,)), ((), ())),
                    preferred_element_type=jnp.float32,
                )
                * SCALE
            )
            m = jnp.max(s, axis=1, keepdims=True)
            p = jnp.exp(s - m)
            l = jnp.sum(p, axis=1, keepdims=True)
            o = (
                lax.dot_general(
                    p, v, (((1,), (0,)), ((), ())),
                    preferred_element_type=jnp.float32,
                )
                / l
            )
            out_ref[0, :, h, :] = o

    return pl.pallas_call(
        body,
        out_shape=jax.ShapeDtypeStruct((B, S_LOC, H, D), jnp.float32),
        in_specs=[
            pl.BlockSpec(memory_space=pltpu.VMEM),
            pl.BlockSpec(memory_space=pltpu.VMEM),
            pl.BlockSpec(memory_space=pltpu.VMEM),
        ],
        out_specs=pl.BlockSpec(memory_space=pltpu.VMEM),
        scratch_shapes=[
            pltpu.VMEM((2, B, S_LOC, H, D), jnp.float32),
            pltpu.SemaphoreType.DMA((2,)),
            pltpu.SemaphoreType.DMA((2,)),
        ],
        compiler_params=pltpu.CompilerParams(collective_id=0),
    )(Q, K, V)


# baseline (device time: 328690 ns/iter reference)
import jax
import jax.numpy as jnp
from jax import lax
from jax.experimental import pallas as pl
from jax.experimental.pallas import tpu as pltpu

B, S_LOC, H, D = 1, 1024, 16, 128
SCALE = D ** -0.5
QB = 256


def kernel(Q, K, V):
    def body(
        q_ref, k_ref, v_ref, out_ref, kv_rem, kv_buf,
        send_sems, recv_sems, copy_sems,
    ):
        my_x = lax.axis_index("x")
        my_y = lax.axis_index("y")
        my_z = lax.axis_index("z")
        nbr = (my_x, 1 - my_y, my_z)

        barrier = pltpu.get_barrier_semaphore()
        pl.semaphore_signal(
            barrier, inc=1, device_id=nbr, device_id_type=pl.DeviceIdType.MESH
        )
        pl.semaphore_wait(barrier, 1)

        rdmas = []
        for t, src in ((0, k_ref), (1, v_ref)):
            r = pltpu.make_async_remote_copy(
                src_ref=src,
                dst_ref=kv_rem.at[t],
                send_sem=send_sems.at[t],
                recv_sem=recv_sems.at[t],
                device_id=nbr,
                device_id_type=pl.DeviceIdType.MESH,
            )
            r.start()
            rdmas.append(r)
        for r in rdmas:
            r.wait()

        def start_fetch(h, slot):
            cps = []
            for t in range(2):
                c = pltpu.make_async_copy(
                    kv_rem.at[t, 0, :, h, :], kv_buf.at[slot, t],
                    copy_sems.at[slot, t],
                )
                c.start()
                cps.append(c)
            return cps

        cps = start_fetch(0, 0)
        for h in range(H):
            slot = h % 2
            for c in cps:
                c.wait()
            if h + 1 < H:
                cps = start_fetch(h + 1, (h + 1) % 2)

            k1 = k_ref[0, :, h, :]
            v1 = v_ref[0, :, h, :]
            k2 = kv_buf[slot, 0]
            v2 = kv_buf[slot, 1]

            def per_qb(i, c, h=h, k1=k1, v1=v1, k2=k2, v2=v2):
                q = q_ref[0, pl.ds(i * QB, QB), h, :]
                s1 = (
                    lax.dot_general(
                        q, k1, (((1,), (1,)), ((), ())),
                        preferred_element_type=jnp.float32,
                    )
                    * SCALE
                )
                s2 = (
                    lax.dot_general(
                        q, k2, (((1,), (1,)), ((), ())),
                        preferred_element_type=jnp.float32,
                    )
                    * SCALE
                )
                m = jnp.maximum(
                    jnp.max(s1, axis=1, keepdims=True),
                    jnp.max(s2, axis=1, keepdims=True),
                )
                p1 = jnp.exp(s1 - m)
                p2 = jnp.exp(s2 - m)
                l = jnp.sum(p1, axis=1, keepdims=True) + jnp.sum(
                    p2, axis=1, keepdims=True
                )
                o = (
                    lax.dot_general(
                        p1, v1, (((1,), (0,)), ((), ())),
                        preferred_element_type=jnp.float32,
                    )
                    + lax.dot_general(
                        p2, v2, (((1,), (0,)), ((), ())),
                        preferred_element_type=jnp.float32,
                    )
                ) / l
                out_ref[0, pl.ds(i * QB, QB), h, :] = o
                return c

            lax.fori_loop(0, S_LOC // QB, per_qb, 0)

    out, _ = pl.pallas_call(
        body,
        out_shape=(
            jax.ShapeDtypeStruct((B, S_LOC, H, D), jnp.float32),
            jax.ShapeDtypeStruct((2, B, S_LOC, H, D), jnp.float32),
        ),
        in_specs=[
            pl.BlockSpec(memory_space=pltpu.MemorySpace.VMEM),
            pl.BlockSpec(memory_space=pltpu.MemorySpace.VMEM),
            pl.BlockSpec(memory_space=pltpu.MemorySpace.VMEM),
        ],
        out_specs=(
            pl.BlockSpec(memory_space=pltpu.MemorySpace.VMEM),
            pl.BlockSpec(memory_space=pltpu.MemorySpace.HBM),
        ),
        scratch_shapes=[
            pltpu.VMEM((2, 2, S_LOC, D), jnp.float32),
            pltpu.SemaphoreType.DMA((2,)),
            pltpu.SemaphoreType.DMA((2,)),
            pltpu.SemaphoreType.DMA((2, 2)),
        ],
        compiler_params=pltpu.CompilerParams(
            collective_id=0, vmem_limit_bytes=63 * 1024 * 1024
        ),
    )(Q, K, V)
    return out


# device time: 171706 ns/iter; 1.9143x vs baseline; 1.9143x over previous
import jax
import jax.numpy as jnp
from jax import lax
from jax.experimental import pallas as pl
from jax.experimental.pallas import tpu as pltpu

B, S_LOC, H, D = 1, 1024, 16, 128
HD = H * D
SCALE = D ** -0.5
QB = 256
QR = S_LOC // 4
CS = QR // 2


def kernel(Q, K, V):
    Q2 = Q.reshape(S_LOC, HD)
    K2 = K.reshape(S_LOC, HD)
    V2 = V.reshape(S_LOC, HD)

    def body(q_ref, k_ref, v_ref, out_ref, kland, l_scr, qs,
             y_send, y_recv, tox_send, fromx_recv, toz_send, fromz_recv):
        my_x = lax.axis_index("x")
        my_y = lax.axis_index("y")
        my_z = lax.axis_index("z")
        y_nbr = (my_x, 1 - my_y, my_z)
        x_nbr = (1 - my_x, my_y, my_z)
        z_nbr = (my_x, my_y, 1 - my_z)

        barrier = pltpu.get_barrier_semaphore()
        for nbr in (y_nbr, x_nbr, z_nbr):
            pl.semaphore_signal(
                barrier, inc=1, device_id=nbr,
                device_id_type=pl.DeviceIdType.MESH,
            )
        pl.semaphore_wait(barrier, 3)

        qi = 2 * my_x + my_z
        xq = 2 * (1 - my_x) + my_z
        zq = 2 * my_x + (1 - my_z)
        dq = 2 * (1 - my_x) + (1 - my_z)

        y_rdmas = []
        for c in range(2):
            rows = pl.ds(qi * QR + c * CS, CS)
            for t, src in ((0, k_ref), (1, v_ref)):
                r = pltpu.make_async_remote_copy(
                    src_ref=src.at[rows, :],
                    dst_ref=kland.at[t, rows, :],
                    send_sem=y_send.at[t, c],
                    recv_sem=y_recv.at[t, c],
                    device_id=y_nbr,
                    device_id_type=pl.DeviceIdType.MESH,
                )
                r.start()
                y_rdmas.append(r)

        qs[:, :] = (q_ref[:, :] * SCALE).astype(jnp.bfloat16)

        for h in range(H):
            lo = h * D
            k1 = k_ref[:, lo:lo + D].astype(jnp.bfloat16)
            v1 = v_ref[:, lo:lo + D].astype(jnp.bfloat16)

            def per_qb(i, c, lo=lo, h=h, k1=k1, v1=v1):
                rq = pl.ds(i * QB, QB)
                q = qs[rq, lo:lo + D]
                s = lax.dot_general(
                    q, k1, (((1,), (1,)), ((), ())),
                    preferred_element_type=jnp.float32,
                )
                p = jnp.exp(s)
                pb = p.astype(jnp.bfloat16)
                l_scr[rq, h:h + 1] = jnp.sum(p, axis=1, keepdims=True)
                out_ref[rq, lo:lo + D] = lax.dot_general(
                    pb, v1, (((1,), (0,)), ((), ())),
                    preferred_element_type=jnp.float32,
                )
                return c

            lax.fori_loop(0, S_LOC // QB, per_qb, 0)

        def chunk_update(row_start, size):
            rows = pl.ds(row_start, size)
            for h in range(H):
                lo = h * D
                kc = kland[0, rows, lo:lo + D].astype(jnp.bfloat16)
                vc = kland[1, rows, lo:lo + D].astype(jnp.bfloat16)
                q = qs[:, lo:lo + D]
                s = lax.dot_general(
                    q, kc, (((1,), (1,)), ((), ())),
                    preferred_element_type=jnp.float32,
                )
                p = jnp.exp(s)
                pb = p.astype(jnp.bfloat16)
                l_scr[:, h:h + 1] = l_scr[:, h:h + 1] + jnp.sum(
                    p, axis=1, keepdims=True
                )
                out_ref[:, lo:lo + D] = out_ref[:, lo:lo + D] + lax.dot_general(
                    pb, vc, (((1,), (0,)), ((), ())),
                    preferred_element_type=jnp.float32,
                )

        def recv_only(t, rows, recv_sem):
            return pltpu.make_async_remote_copy(
                src_ref=kland.at[t, rows, :],
                dst_ref=kland.at[t, rows, :],
                send_sem=y_send.at[t, 0],
                recv_sem=recv_sem,
                device_id=z_nbr,
                device_id_type=pl.DeviceIdType.MESH,
            )

        fwds = []

        for c in range(2):
            rows = pl.ds(qi * QR + c * CS, CS)
            for t in range(2):
                y_rdmas[2 * c + t].wait_recv()
                for nbr, ss, rs in (
                    (x_nbr, tox_send.at[t, c], fromx_recv.at[t, c]),
                    (z_nbr, toz_send.at[t, c], fromz_recv.at[t, c]),
                ):
                    f = pltpu.make_async_remote_copy(
                        src_ref=kland.at[t, rows, :],
                        dst_ref=kland.at[t, rows, :],
                        send_sem=ss,
                        recv_sem=rs,
                        device_id=nbr,
                        device_id_type=pl.DeviceIdType.MESH,
                    )
                    f.start()
                    fwds.append(f)
            chunk_update(qi * QR + c * CS, CS)

        for c in range(2):
            rows = pl.ds(xq * QR + c * CS, CS)
            for t in range(2):
                recv_only(t, rows, fromx_recv.at[t, c]).wait_recv()
                if c == 1:
                    f = pltpu.make_async_remote_copy(
                        src_ref=kland.at[t, rows, :],
                        dst_ref=kland.at[t, rows, :],
                        send_sem=toz_send.at[t, 2],
                        recv_sem=fromz_recv.at[t, 2],
                        device_id=z_nbr,
                        device_id_type=pl.DeviceIdType.MESH,
                    )
                    f.start()
                    fwds.append(f)
            chunk_update(xq * QR + c * CS, CS)

        for c in range(2):
            rows = pl.ds(zq * QR + c * CS, CS)
            for t in range(2):
                recv_only(t, rows, fromz_recv.at[t, c]).wait_recv()
                if c == 0:
                    f = pltpu.make_async_remote_copy(
                        src_ref=kland.at[t, rows, :],
                        dst_ref=kland.at[t, rows, :],
                        send_sem=tox_send.at[t, 2],
                        recv_sem=fromx_recv.at[t, 2],
                        device_id=x_nbr,
                        device_id_type=pl.DeviceIdType.MESH,
                    )
                    f.start()
                    fwds.append(f)
            chunk_update(zq * QR + c * CS, CS)

        for t in range(2):
            recv_only(t, pl.ds(dq * QR, CS), fromx_recv.at[t, 2]).wait_recv()
        chunk_update(dq * QR, CS)
        for t in range(2):
            recv_only(t, pl.ds(dq * QR + CS, CS), fromz_recv.at[t, 2]).wait_recv()
        rows = pl.ds(dq * QR + CS, CS)
        for h in range(H):
            lo = h * D
            kc = kland[0, rows, lo:lo + D].astype(jnp.bfloat16)
            vc = kland[1, rows, lo:lo + D].astype(jnp.bfloat16)
            q = qs[:, lo:lo + D]
            s = lax.dot_general(
                q, kc, (((1,), (1,)), ((), ())),
                preferred_element_type=jnp.float32,
            )
            p = jnp.exp(s)
            pb = p.astype(jnp.bfloat16)
            l = l_scr[:, h:h + 1] + jnp.sum(p, axis=1, keepdims=True)
            out_ref[:, lo:lo + D] = (
                out_ref[:, lo:lo + D]
                + lax.dot_general(
                    pb, vc, (((1,), (0,)), ((), ())),
                    preferred_element_type=jnp.float32,
                )
            ) / l

        for r in y_rdmas:
            r.wait_send()
        for f in fwds:
            f.wait_send()

    out = pl.pallas_call(
        body,
        out_shape=jax.ShapeDtypeStruct((S_LOC, HD), jnp.float32),
        in_specs=[
            pl.BlockSpec(memory_space=pltpu.MemorySpace.VMEM),
            pl.BlockSpec(memory_space=pltpu.MemorySpace.VMEM),
            pl.BlockSpec(memory_space=pltpu.MemorySpace.VMEM),
        ],
        out_specs=pl.BlockSpec(memory_space=pltpu.MemorySpace.VMEM),
        scratch_shapes=[
            pltpu.VMEM((2, S_LOC, HD), jnp.float32),
            pltpu.VMEM((S_LOC, 128), jnp.float32),
            pltpu.VMEM((S_LOC, HD), jnp.bfloat16),
            pltpu.SemaphoreType.DMA((2, 2)),
            pltpu.SemaphoreType.DMA((2, 2)),
            pltpu.SemaphoreType.DMA((2, 3)),
            pltpu.SemaphoreType.DMA((2, 3)),
            pltpu.SemaphoreType.DMA((2, 3)),
            pltpu.SemaphoreType.DMA((2, 3)),
        ],
        compiler_params=pltpu.CompilerParams(
            collective_id=0, vmem_limit_bytes=63 * 1024 * 1024
        ),
    )(Q2, K2, V2)
    return out.reshape(B, S_LOC, H, D)
